# baseline (speedup 1.0000x reference)
import functools

import jax
import jax.numpy as jnp
from jax import lax
from jax.experimental import pallas as pl
from jax.experimental.pallas import tpu as pltpu
from jax.experimental.pallas import tpu_sc as plsc

CLS_NUM = 100000
FEATURE_NUM = 64
BATCH = 16384

NC = 2
NS = 16
NW = NC * NS
B_PER_W = BATCH // NW
ROWS_2D = BATCH // 128
HIST_PAD = 100096
HIST_PER_TILE = HIST_PAD // NS


def _sc_body(labels_hbm, zeros_hbm, center_hbm, rows_hbm, cnt_hbm,
             labv_my, labv_hist, ones_v, cntv, rows_v, hist, sem):
    cid = lax.axis_index("c")
    sid = lax.axis_index("s")
    wid = sid * NC + cid

    pltpu.sync_copy(labels_hbm.at[pl.ds(wid * 4, 4)], labv_my)
    cps = [
        pltpu.async_copy(
            center_hbm.at[labv_my.at[k]],
            rows_v.at[pl.ds(k * 128, 128)],
            sem,
        )
        for k in range(4)
    ]

    pltpu.sync_copy(
        zeros_hbm.at[pl.ds(sid * HIST_PER_TILE, HIST_PER_TILE)],
        hist.at[pl.ds(sid * HIST_PER_TILE, HIST_PER_TILE)],
    )

    for j in range(8):
        ones_v[pl.ds(j * 16, 16)] = jnp.ones((16,), jnp.float32)

    pltpu.sync_copy(labels_hbm.at[pl.ds(sid * 8, 8)], labv_hist)

    plsc.subcore_barrier()

    for k in range(8):
        pltpu.sync_copy(ones_v, hist.at[labv_hist.at[k]], add=True)

    plsc.subcore_barrier()

    for k in range(4):
        pltpu.sync_copy(hist.at[labv_my.at[k]], cntv.at[pl.ds(k * 128, 128)])
    pltpu.sync_copy(cntv, cnt_hbm.at[pl.ds(wid * B_PER_W, B_PER_W)])

    for cp in cps:
        cp.wait()
    pltpu.sync_copy(rows_v, rows_hbm.at[pl.ds(wid * B_PER_W, B_PER_W)])


_sc_gather = functools.partial(
    pl.kernel,
    mesh=plsc.VectorSubcoreMesh(core_axis_name="c", subcore_axis_name="s"),
    compiler_params=pltpu.CompilerParams(use_tc_tiling_on_sc=False),
    out_type=[
        jax.ShapeDtypeStruct((BATCH, FEATURE_NUM), jnp.float32),
        jax.ShapeDtypeStruct((BATCH,), jnp.float32),
    ],
    scratch_types=[
        pltpu.VMEM((4, 128), jnp.int32),
        pltpu.VMEM((8, 128), jnp.int32),
        pltpu.VMEM((128,), jnp.float32),
        pltpu.VMEM((B_PER_W,), jnp.float32),
        pltpu.VMEM((B_PER_W, FEATURE_NUM), jnp.float32),
        pltpu.VMEM_SHARED((HIST_PAD,), jnp.float32),
        pltpu.SemaphoreType.DMA,
    ],
)(_sc_body)


def _tc_body(datas_ref, rows_ref, cnt_ref, out_ref):
    d = datas_ref[...] - rows_ref[...]
    d2 = jnp.sum(d * d, axis=1, keepdims=True)
    out_ref[...] = jnp.sum(jnp.sqrt(d2) / cnt_ref[...]).reshape(1, 1)


_tc_tail = pl.pallas_call(
    _tc_body,
    out_shape=jax.ShapeDtypeStruct((1, 1), jnp.float32),
)


@jax.jit
def kernel(datas, labels, center):
    labels2d = labels.astype(jnp.int32).reshape(ROWS_2D, 128)
    zeros = jnp.zeros((HIST_PAD,), jnp.float32)
    rows, cnt = _sc_gather(labels2d, zeros, center)
    out = _tc_tail(datas, rows, cnt.reshape(BATCH, 1))
    return out[0, 0]
